# baseline (speedup 1.0000x reference)
import functools

import jax
import jax.numpy as jnp
from jax import lax
from jax.experimental import pallas as pl
from jax.experimental.pallas import tpu as pltpu
from jax.experimental.pallas import tpu_sc as plsc

D_MODEL = 768
N_EXP = 64
HALF = N_EXP // 2
TEMP = 0.1
N_TOK = 32768
BT = 4096

_info = plsc.get_sparse_core_info()
_NC, _NS, _L = _info.num_cores, _info.num_subcores, _info.num_lanes
_NW = _NC * _NS
TOK_W = N_TOK // _NW
_GRP = TOK_W // _L
_UNROLL = 8


def _logits_body(x_ref, wt_ref, b_ref, logits_ref, pmax_ref, pminb_ref):
    logits = (
        jnp.dot(x_ref[...], wt_ref[...], preferred_element_type=jnp.float32)
        + b_ref[...][None, :]) / TEMP
    logits_ref[...] = logits
    lt = logits.T
    a = lt[:HALF, :]
    b2 = lt[HALF:, :]
    pmax = jnp.maximum(a, b2)
    pmin = jnp.minimum(a, b2)
    bit = (a >= b2).astype(jnp.int32)
    pmin_i = lax.bitcast_convert_type(pmin, jnp.int32)
    pmax_ref[...] = pmax
    pminb_ref[...] = lax.bitcast_convert_type((pmin_i & -2) | bit, jnp.float32)


_sc_mesh = plsc.VectorSubcoreMesh(core_axis_name="c", subcore_axis_name="s")


@functools.partial(
    pl.kernel,
    mesh=_sc_mesh,
    out_type=[
        jax.ShapeDtypeStruct((2, N_TOK), jnp.float32),
        jax.ShapeDtypeStruct((2, N_TOK), jnp.int32),
    ],
    scratch_types=[
        pltpu.VMEM((HALF, TOK_W), jnp.float32),
        pltpu.VMEM((HALF, TOK_W), jnp.float32),
        pltpu.VMEM((TOK_W,), jnp.float32),
        pltpu.VMEM((TOK_W,), jnp.float32),
        pltpu.VMEM((TOK_W,), jnp.int32),
        pltpu.VMEM((TOK_W,), jnp.int32),
    ],
)
def _sc_topk(pmax_hbm, pminb_hbm, probs_hbm, idx_hbm,
             pmax_v, pmb_v, p1_v, p2_v, i1_v, i2_v):
    wid = lax.axis_index("s") * _NC + lax.axis_index("c")
    base = wid * TOK_W
    pltpu.sync_copy(pmax_hbm.at[:, pl.ds(base, TOK_W)], pmax_v)
    pltpu.sync_copy(pminb_hbm.at[:, pl.ds(base, TOK_W)], pmb_v)

    neg = jnp.full((_L,), -jnp.inf, jnp.float32)
    zero = jnp.zeros((_L,), jnp.int32)
    zerof = jnp.zeros((_L,), jnp.float32)

    def super_group(sg, _):
        offs = [sg * (_UNROLL * _L) + g * _L for g in range(_UNROLL)]
        m1 = [neg] * _UNROLL
        m2 = [neg] * _UNROLL
        j1p = [zero] * _UNROLL
        j2p = [zero] * _UNROLL
        winm = [zerof] * _UNROLL
        renc = [zerof] * _UNROLL
        for p in range(HALF):
            pi = jnp.full((_L,), p, jnp.int32)
            for g in range(_UNROLL):
                v = pmax_v[p, pl.ds(offs[g], _L)]
                w = pmb_v[p, pl.ds(offs[g], _L)]
                gt1 = v > m1[g]
                lose = jnp.minimum(v, m1[g])
                gt2 = lose > m2[g]
                nj1 = jnp.where(gt1, pi, j1p[g])
                tjp = jnp.where(gt1, j1p[g], pi)
                twe = jnp.where(gt1, winm[g], w)
                j2p[g] = jnp.where(gt2, tjp, j2p[g])
                renc[g] = jnp.where(gt2, twe, renc[g])
                winm[g] = jnp.where(gt1, w, winm[g])
                m1[g] = jnp.maximum(v, m1[g])
                m2[g] = jnp.maximum(lose, m2[g])
                j1p[g] = nj1
        for g in range(_UNROLL):
            wi = lax.bitcast_convert_type(winm[g], jnp.int32)
            bit1 = wi & 1
            wval = lax.bitcast_convert_type(wi & -2, jnp.float32)
            ri = lax.bitcast_convert_type(renc[g], jnp.int32)
            bit2 = ri & 1
            j1 = j1p[g] + (1 - bit1) * HALF
            loser = j1p[g] + bit1 * HALF
            j2cand = j2p[g] + (1 - bit2) * HALF
            gtw = wval > m2[g]
            eqw = wval == m2[g]
            m2v = jnp.maximum(wval, m2[g])
            j2 = jnp.where(
                gtw, loser,
                jnp.where(eqw, jnp.minimum(loser, j2cand), j2cand))
            e2 = jnp.exp(m2v - m1[g])
            p1 = 1.0 / (1.0 + e2)
            p1_v[pl.ds(offs[g], _L)] = p1
            p2_v[pl.ds(offs[g], _L)] = e2 * p1
            i1_v[pl.ds(offs[g], _L)] = j1
            i2_v[pl.ds(offs[g], _L)] = j2
        return 0

    lax.fori_loop(0, _GRP // _UNROLL, super_group, 0)

    pltpu.sync_copy(p1_v, probs_hbm.at[0, pl.ds(base, TOK_W)])
    pltpu.sync_copy(p2_v, probs_hbm.at[1, pl.ds(base, TOK_W)])
    pltpu.sync_copy(i1_v, idx_hbm.at[0, pl.ds(base, TOK_W)])
    pltpu.sync_copy(i2_v, idx_hbm.at[1, pl.ds(base, TOK_W)])


@jax.jit
def kernel(x, W, b):
    n_tokens = x.shape[0]
    wt = W.T
    logits, pmax, pminb = pl.pallas_call(
        _logits_body,
        grid=(n_tokens // BT,),
        in_specs=[
            pl.BlockSpec((BT, D_MODEL), lambda i: (i, 0)),
            pl.BlockSpec((D_MODEL, N_EXP), lambda i: (0, 0)),
            pl.BlockSpec((N_EXP,), lambda i: (0,)),
        ],
        out_specs=[
            pl.BlockSpec((BT, N_EXP), lambda i: (i, 0)),
            pl.BlockSpec((HALF, BT), lambda i: (0, i)),
            pl.BlockSpec((HALF, BT), lambda i: (0, i)),
        ],
        out_shape=[
            jax.ShapeDtypeStruct((n_tokens, N_EXP), jnp.float32),
            jax.ShapeDtypeStruct((HALF, n_tokens), jnp.float32),
            jax.ShapeDtypeStruct((HALF, n_tokens), jnp.float32),
        ],
    )(x, wt, b)
    probs_t, idx_t = _sc_topk(pmax, pminb)
    return (logits, probs_t.T, idx_t.T)


# speedup vs baseline: 1.0063x; 1.0063x over previous
import functools

import jax
import jax.numpy as jnp
from jax import lax
from jax.experimental import pallas as pl
from jax.experimental.pallas import tpu as pltpu
from jax.experimental.pallas import tpu_sc as plsc

D_MODEL = 768
N_EXP = 64
HALF = N_EXP // 2
TEMP = 0.1
N_TOK = 32768
BT = 4096

_info = plsc.get_sparse_core_info()
_NC, _NS, _L = _info.num_cores, _info.num_subcores, _info.num_lanes
_NW = _NC * _NS
TOK_W = N_TOK // _NW
_GRP = TOK_W // _L
_UNROLL = 2


def _logits_body(x_ref, wt_ref, b_ref, logits_ref, pmax_ref, pminb_ref):
    logits = (
        jnp.dot(x_ref[...], wt_ref[...], preferred_element_type=jnp.float32)
        + b_ref[...][None, :]) / TEMP
    logits_ref[...] = logits
    lt = logits.T
    a = lt[:HALF, :]
    b2 = lt[HALF:, :]
    pmax = jnp.maximum(a, b2)
    pmin = jnp.minimum(a, b2)
    bit = (a >= b2).astype(jnp.int32)
    pmin_i = lax.bitcast_convert_type(pmin, jnp.int32)
    pmax_ref[...] = pmax
    pminb_ref[...] = lax.bitcast_convert_type((pmin_i & -2) | bit, jnp.float32)


_sc_mesh = plsc.VectorSubcoreMesh(core_axis_name="c", subcore_axis_name="s")


@functools.partial(
    pl.kernel,
    mesh=_sc_mesh,
    out_type=[
        jax.ShapeDtypeStruct((2, N_TOK), jnp.float32),
        jax.ShapeDtypeStruct((2, N_TOK), jnp.int32),
    ],
    scratch_types=[
        pltpu.VMEM((HALF, TOK_W), jnp.float32),
        pltpu.VMEM((HALF, TOK_W), jnp.float32),
        pltpu.VMEM((TOK_W,), jnp.float32),
        pltpu.VMEM((TOK_W,), jnp.float32),
        pltpu.VMEM((TOK_W,), jnp.int32),
        pltpu.VMEM((TOK_W,), jnp.int32),
    ],
)
def _sc_topk(pmax_hbm, pminb_hbm, probs_hbm, idx_hbm,
             pmax_v, pmb_v, p1_v, p2_v, i1_v, i2_v):
    wid = lax.axis_index("s") * _NC + lax.axis_index("c")
    base = wid * TOK_W
    pltpu.sync_copy(pmax_hbm.at[:, pl.ds(base, TOK_W)], pmax_v)
    pltpu.sync_copy(pminb_hbm.at[:, pl.ds(base, TOK_W)], pmb_v)

    neg = jnp.full((_L,), -jnp.inf, jnp.float32)
    zero = jnp.zeros((_L,), jnp.int32)
    zerof = jnp.zeros((_L,), jnp.float32)

    def super_group(sg, _):
        offs = [sg * (_UNROLL * _L) + g * _L for g in range(_UNROLL)]
        m1 = [neg] * _UNROLL
        m2 = [neg] * _UNROLL
        j1p = [zero] * _UNROLL
        j2p = [zero] * _UNROLL
        winm = [zerof] * _UNROLL
        renc = [zerof] * _UNROLL
        for p in range(HALF):
            pi = jnp.full((_L,), p, jnp.int32)
            for g in range(_UNROLL):
                v = pmax_v[p, pl.ds(offs[g], _L)]
                w = pmb_v[p, pl.ds(offs[g], _L)]
                gt1 = v > m1[g]
                lose = jnp.minimum(v, m1[g])
                gt2 = lose > m2[g]
                nj1 = jnp.where(gt1, pi, j1p[g])
                tjp = jnp.where(gt1, j1p[g], pi)
                twe = jnp.where(gt1, winm[g], w)
                j2p[g] = jnp.where(gt2, tjp, j2p[g])
                renc[g] = jnp.where(gt2, twe, renc[g])
                winm[g] = jnp.where(gt1, w, winm[g])
                m1[g] = jnp.maximum(v, m1[g])
                m2[g] = jnp.maximum(lose, m2[g])
                j1p[g] = nj1
        for g in range(_UNROLL):
            wi = lax.bitcast_convert_type(winm[g], jnp.int32)
            bit1 = wi & 1
            wval = lax.bitcast_convert_type(wi & -2, jnp.float32)
            ri = lax.bitcast_convert_type(renc[g], jnp.int32)
            bit2 = ri & 1
            j1 = j1p[g] + (1 - bit1) * HALF
            loser = j1p[g] + bit1 * HALF
            j2cand = j2p[g] + (1 - bit2) * HALF
            gtw = wval > m2[g]
            eqw = wval == m2[g]
            m2v = jnp.maximum(wval, m2[g])
            j2 = jnp.where(
                gtw, loser,
                jnp.where(eqw, jnp.minimum(loser, j2cand), j2cand))
            e2 = jnp.exp(m2v - m1[g])
            p1 = 1.0 / (1.0 + e2)
            p1_v[pl.ds(offs[g], _L)] = p1
            p2_v[pl.ds(offs[g], _L)] = e2 * p1
            i1_v[pl.ds(offs[g], _L)] = j1
            i2_v[pl.ds(offs[g], _L)] = j2
        return 0

    lax.fori_loop(0, _GRP // _UNROLL, super_group, 0)

    pltpu.sync_copy(p1_v, probs_hbm.at[0, pl.ds(base, TOK_W)])
    pltpu.sync_copy(p2_v, probs_hbm.at[1, pl.ds(base, TOK_W)])
    pltpu.sync_copy(i1_v, idx_hbm.at[0, pl.ds(base, TOK_W)])
    pltpu.sync_copy(i2_v, idx_hbm.at[1, pl.ds(base, TOK_W)])


@jax.jit
def kernel(x, W, b):
    n_tokens = x.shape[0]
    wt = W.T
    logits, pmax, pminb = pl.pallas_call(
        _logits_body,
        grid=(n_tokens // BT,),
        in_specs=[
            pl.BlockSpec((BT, D_MODEL), lambda i: (i, 0)),
            pl.BlockSpec((D_MODEL, N_EXP), lambda i: (0, 0)),
            pl.BlockSpec((N_EXP,), lambda i: (0,)),
        ],
        out_specs=[
            pl.BlockSpec((BT, N_EXP), lambda i: (i, 0)),
            pl.BlockSpec((HALF, BT), lambda i: (0, i)),
            pl.BlockSpec((HALF, BT), lambda i: (0, i)),
        ],
        out_shape=[
            jax.ShapeDtypeStruct((n_tokens, N_EXP), jnp.float32),
            jax.ShapeDtypeStruct((HALF, n_tokens), jnp.float32),
            jax.ShapeDtypeStruct((HALF, n_tokens), jnp.float32),
        ],
    )(x, wt, b)
    probs_t, idx_t = _sc_topk(pmax, pminb)
    return (logits, probs_t.T, idx_t.T)


# speedup vs baseline: 1.0070x; 1.0007x over previous
import functools

import jax
import jax.numpy as jnp
from jax import lax
from jax.experimental import pallas as pl
from jax.experimental.pallas import tpu as pltpu
from jax.experimental.pallas import tpu_sc as plsc

D_MODEL = 768
N_EXP = 64
HALF = N_EXP // 2
TEMP = 0.1
N_TOK = 32768
BT = 4096

_info = plsc.get_sparse_core_info()
_NC, _NS, _L = _info.num_cores, _info.num_subcores, _info.num_lanes
_NW = _NC * _NS
TOK_W = N_TOK // _NW
_GRP = TOK_W // _L
_UNROLL = 4
_SPANS = BT // TOK_W


def _logits_body(x_ref, wt_ref, b_ref, logits_ref, pmax_ref, pminb_ref):
    logits = (
        jnp.dot(x_ref[...], wt_ref[...], preferred_element_type=jnp.float32)
        + b_ref[...][None, :]) / TEMP
    logits_ref[...] = logits
    lt = logits.T
    a = lt[:HALF, :]
    b2 = lt[HALF:, :]
    pmax = jnp.maximum(a, b2)
    pmin = jnp.minimum(a, b2)
    bit = (a >= b2).astype(jnp.int32)
    pmin_i = lax.bitcast_convert_type(pmin, jnp.int32)
    pminb = lax.bitcast_convert_type((pmin_i & -2) | bit, jnp.float32)
    pmax_ref[...] = pmax.reshape(HALF, _SPANS, TOK_W).transpose(1, 0, 2)
    pminb_ref[...] = pminb.reshape(HALF, _SPANS, TOK_W).transpose(1, 0, 2)


_sc_mesh = plsc.VectorSubcoreMesh(core_axis_name="c", subcore_axis_name="s")


@functools.partial(
    pl.kernel,
    mesh=_sc_mesh,
    out_type=[
        jax.ShapeDtypeStruct((2, N_TOK), jnp.float32),
        jax.ShapeDtypeStruct((2, N_TOK), jnp.int32),
    ],
    scratch_types=[
        pltpu.VMEM((HALF, TOK_W), jnp.float32),
        pltpu.VMEM((HALF, TOK_W), jnp.float32),
        pltpu.VMEM((TOK_W,), jnp.float32),
        pltpu.VMEM((TOK_W,), jnp.float32),
        pltpu.VMEM((TOK_W,), jnp.int32),
        pltpu.VMEM((TOK_W,), jnp.int32),
    ],
)
def _sc_topk(pmax_hbm, pminb_hbm, probs_hbm, idx_hbm,
             pmax_v, pmb_v, p1_v, p2_v, i1_v, i2_v):
    wid = lax.axis_index("s") * _NC + lax.axis_index("c")
    base = wid * TOK_W
    pltpu.sync_copy(pmax_hbm.at[wid], pmax_v)
    pltpu.sync_copy(pminb_hbm.at[wid], pmb_v)

    neg = jnp.full((_L,), -jnp.inf, jnp.float32)
    zero = jnp.zeros((_L,), jnp.int32)
    zerof = jnp.zeros((_L,), jnp.float32)

    def super_group(sg, _):
        offs = [sg * (_UNROLL * _L) + g * _L for g in range(_UNROLL)]
        m1 = [neg] * _UNROLL
        m2 = [neg] * _UNROLL
        j1p = [zero] * _UNROLL
        j2p = [zero] * _UNROLL
        winm = [zerof] * _UNROLL
        renc = [zerof] * _UNROLL
        for p in range(HALF):
            pi = jnp.full((_L,), p, jnp.int32)
            for g in range(_UNROLL):
                v = pmax_v[p, pl.ds(offs[g], _L)]
                w = pmb_v[p, pl.ds(offs[g], _L)]
                gt1 = v > m1[g]
                lose = jnp.minimum(v, m1[g])
                gt2 = lose > m2[g]
                nj1 = jnp.where(gt1, pi, j1p[g])
                tjp = jnp.where(gt1, j1p[g], pi)
                twe = jnp.where(gt1, winm[g], w)
                j2p[g] = jnp.where(gt2, tjp, j2p[g])
                renc[g] = jnp.where(gt2, twe, renc[g])
                winm[g] = jnp.where(gt1, w, winm[g])
                m1[g] = jnp.maximum(v, m1[g])
                m2[g] = jnp.maximum(lose, m2[g])
                j1p[g] = nj1
        for g in range(_UNROLL):
            wi = lax.bitcast_convert_type(winm[g], jnp.int32)
            bit1 = wi & 1
            wval = lax.bitcast_convert_type(wi & -2, jnp.float32)
            ri = lax.bitcast_convert_type(renc[g], jnp.int32)
            bit2 = ri & 1
            j1 = j1p[g] + (1 - bit1) * HALF
            loser = j1p[g] + bit1 * HALF
            j2cand = j2p[g] + (1 - bit2) * HALF
            gtw = wval > m2[g]
            eqw = wval == m2[g]
            m2v = jnp.maximum(wval, m2[g])
            j2 = jnp.where(
                gtw, loser,
                jnp.where(eqw, jnp.minimum(loser, j2cand), j2cand))
            e2 = jnp.exp(m2v - m1[g])
            p1 = 1.0 / (1.0 + e2)
            p1_v[pl.ds(offs[g], _L)] = p1
            p2_v[pl.ds(offs[g], _L)] = e2 * p1
            i1_v[pl.ds(offs[g], _L)] = j1
            i2_v[pl.ds(offs[g], _L)] = j2
        return 0

    lax.fori_loop(0, _GRP // _UNROLL, super_group, 0)

    pltpu.sync_copy(p1_v, probs_hbm.at[0, pl.ds(base, TOK_W)])
    pltpu.sync_copy(p2_v, probs_hbm.at[1, pl.ds(base, TOK_W)])
    pltpu.sync_copy(i1_v, idx_hbm.at[0, pl.ds(base, TOK_W)])
    pltpu.sync_copy(i2_v, idx_hbm.at[1, pl.ds(base, TOK_W)])


@jax.jit
def kernel(x, W, b):
    n_tokens = x.shape[0]
    wt = W.T
    logits, pmax, pminb = pl.pallas_call(
        _logits_body,
        grid=(n_tokens // BT,),
        in_specs=[
            pl.BlockSpec((BT, D_MODEL), lambda i: (i, 0)),
            pl.BlockSpec((D_MODEL, N_EXP), lambda i: (0, 0)),
            pl.BlockSpec((N_EXP,), lambda i: (0,)),
        ],
        out_specs=[
            pl.BlockSpec((BT, N_EXP), lambda i: (i, 0)),
            pl.BlockSpec((_SPANS, HALF, TOK_W), lambda i: (i, 0, 0)),
            pl.BlockSpec((_SPANS, HALF, TOK_W), lambda i: (i, 0, 0)),
        ],
        out_shape=[
            jax.ShapeDtypeStruct((n_tokens, N_EXP), jnp.float32),
            jax.ShapeDtypeStruct((_NW, HALF, TOK_W), jnp.float32),
            jax.ShapeDtypeStruct((_NW, HALF, TOK_W), jnp.float32),
        ],
    )(x, wt, b)
    probs_t, idx_t = _sc_topk(pmax, pminb)
    return (logits, probs_t.T, idx_t.T)


# speedup vs baseline: 1.0372x; 1.0299x over previous
import functools

import jax
import jax.numpy as jnp
from jax import lax
from jax.experimental import pallas as pl
from jax.experimental.pallas import tpu as pltpu
from jax.experimental.pallas import tpu_sc as plsc

D_MODEL = 768
N_EXP = 64
NQ = N_EXP // 4
TEMP = 0.1
N_TOK = 32768
BT = 4096

_info = plsc.get_sparse_core_info()
_NC, _NS, _L = _info.num_cores, _info.num_subcores, _info.num_lanes
_NW = _NC * _NS
TOK_W = N_TOK // _NW
_GRP = TOK_W // _L
_UNROLL = 4
_SPANS = BT // TOK_W


def _enc(v, pos):
    return lax.bitcast_convert_type(
        (lax.bitcast_convert_type(v, jnp.int32) & -4) | pos, jnp.float32)


def _logits_body(x_ref, wt_ref, b_ref, logits_ref, qmax_ref, qsec_ref):
    logits = (
        jnp.dot(x_ref[...], wt_ref[...], preferred_element_type=jnp.float32)
        + b_ref[...][None, :]) / TEMP
    logits_ref[...] = logits
    lt = logits.T
    c0 = lt[0 * NQ:1 * NQ]
    c1 = lt[1 * NQ:2 * NQ]
    c2 = lt[2 * NQ:3 * NQ]
    c3 = lt[3 * NQ:4 * NQ]
    i32 = jnp.int32
    amax = jnp.maximum(c0, c2)
    amin = jnp.minimum(c0, c2)
    abit = jnp.where(c0 >= c2, i32(0), i32(2))
    bmax = jnp.maximum(c1, c3)
    bmin = jnp.minimum(c1, c3)
    bbit = jnp.where(c1 >= c3, i32(1), i32(3))
    awin = (amax > bmax) | ((amax == bmax) & (abit < bbit))
    qmax = jnp.where(awin, amax, bmax)
    qmaxp = jnp.where(awin, abit, bbit)
    lmax = jnp.where(awin, bmax, amax)
    lmaxp = jnp.where(awin, bbit, abit)
    wmin = jnp.where(awin, amin, bmin)
    wminp = jnp.where(awin, 2 - abit, 4 - bbit)
    sec_is_l = (lmax > wmin) | ((lmax == wmin) & (lmaxp < wminp))
    qsec = jnp.where(sec_is_l, lmax, wmin)
    qsecp = jnp.where(sec_is_l, lmaxp, wminp)
    qmax_ref[...] = _enc(qmax, qmaxp).reshape(NQ, _SPANS, TOK_W).transpose(1, 0, 2)
    qsec_ref[...] = _enc(qsec, qsecp).reshape(NQ, _SPANS, TOK_W).transpose(1, 0, 2)


_sc_mesh = plsc.VectorSubcoreMesh(core_axis_name="c", subcore_axis_name="s")


@functools.partial(
    pl.kernel,
    mesh=_sc_mesh,
    out_type=[
        jax.ShapeDtypeStruct((2, N_TOK), jnp.float32),
        jax.ShapeDtypeStruct((2, N_TOK), jnp.int32),
    ],
    scratch_types=[
        pltpu.VMEM((NQ, TOK_W), jnp.float32),
        pltpu.VMEM((NQ, TOK_W), jnp.float32),
        pltpu.VMEM((TOK_W,), jnp.float32),
        pltpu.VMEM((TOK_W,), jnp.float32),
        pltpu.VMEM((TOK_W,), jnp.int32),
        pltpu.VMEM((TOK_W,), jnp.int32),
    ],
)
def _sc_topk(qmax_hbm, qsec_hbm, probs_hbm, idx_hbm,
             qmax_v, qsec_v, p1_v, p2_v, i1_v, i2_v):
    wid = lax.axis_index("s") * _NC + lax.axis_index("c")
    base = wid * TOK_W
    pltpu.sync_copy(qmax_hbm.at[wid], qmax_v)
    pltpu.sync_copy(qsec_hbm.at[wid], qsec_v)

    neg = jnp.full((_L,), -jnp.inf, jnp.float32)
    zero = jnp.zeros((_L,), jnp.int32)
    zerof = jnp.zeros((_L,), jnp.float32)

    def super_group(sg, _):
        offs = [sg * (_UNROLL * _L) + g * _L for g in range(_UNROLL)]
        m1 = [neg] * _UNROLL
        m2 = [neg] * _UNROLL
        j1q = [zero] * _UNROLL
        j2q = [zero] * _UNROLL
        wsec = [zerof] * _UNROLL
        for q in range(NQ):
            qi = jnp.full((_L,), q, jnp.int32)
            for g in range(_UNROLL):
                v = qmax_v[q, pl.ds(offs[g], _L)]
                s = qsec_v[q, pl.ds(offs[g], _L)]
                gt1 = v > m1[g]
                lose = jnp.minimum(v, m1[g])
                gt2 = lose > m2[g]
                nj1 = jnp.where(gt1, qi, j1q[g])
                tjq = jnp.where(gt1, j1q[g], qi)
                j2q[g] = jnp.where(gt2, tjq, j2q[g])
                wsec[g] = jnp.where(gt1, s, wsec[g])
                m1[g] = jnp.maximum(v, m1[g])
                m2[g] = jnp.maximum(lose, m2[g])
                j1q[g] = nj1
        for g in range(_UNROLL):
            mi = lax.bitcast_convert_type(m1[g], jnp.int32)
            m1v = lax.bitcast_convert_type(mi & -4, jnp.float32)
            j1 = j1q[g] + (mi & 3) * NQ
            wi = lax.bitcast_convert_type(wsec[g], jnp.int32)
            wsecv = lax.bitcast_convert_type(wi & -4, jnp.float32)
            wsecidx = j1q[g] + (wi & 3) * NQ
            ri = lax.bitcast_convert_type(m2[g], jnp.int32)
            rmaxv = lax.bitcast_convert_type(ri & -4, jnp.float32)
            rmaxidx = j2q[g] + (ri & 3) * NQ
            gtw = wsecv > rmaxv
            eqw = wsecv == rmaxv
            m2v = jnp.maximum(wsecv, rmaxv)
            j2 = jnp.where(
                gtw, wsecidx,
                jnp.where(eqw, jnp.minimum(wsecidx, rmaxidx), rmaxidx))
            e2 = jnp.exp(m2v - m1v)
            p1 = 1.0 / (1.0 + e2)
            p1_v[pl.ds(offs[g], _L)] = p1
            p2_v[pl.ds(offs[g], _L)] = e2 * p1
            i1_v[pl.ds(offs[g], _L)] = j1
            i2_v[pl.ds(offs[g], _L)] = j2
        return 0

    lax.fori_loop(0, _GRP // _UNROLL, super_group, 0)

    pltpu.sync_copy(p1_v, probs_hbm.at[0, pl.ds(base, TOK_W)])
    pltpu.sync_copy(p2_v, probs_hbm.at[1, pl.ds(base, TOK_W)])
    pltpu.sync_copy(i1_v, idx_hbm.at[0, pl.ds(base, TOK_W)])
    pltpu.sync_copy(i2_v, idx_hbm.at[1, pl.ds(base, TOK_W)])


@jax.jit
def kernel(x, W, b):
    n_tokens = x.shape[0]
    wt = W.T
    logits, qmax, qsec = pl.pallas_call(
        _logits_body,
        grid=(n_tokens // BT,),
        in_specs=[
            pl.BlockSpec((BT, D_MODEL), lambda i: (i, 0)),
            pl.BlockSpec((D_MODEL, N_EXP), lambda i: (0, 0)),
            pl.BlockSpec((N_EXP,), lambda i: (0,)),
        ],
        out_specs=[
            pl.BlockSpec((BT, N_EXP), lambda i: (i, 0)),
            pl.BlockSpec((_SPANS, NQ, TOK_W), lambda i: (i, 0, 0)),
            pl.BlockSpec((_SPANS, NQ, TOK_W), lambda i: (i, 0, 0)),
        ],
        out_shape=[
            jax.ShapeDtypeStruct((n_tokens, N_EXP), jnp.float32),
            jax.ShapeDtypeStruct((_NW, NQ, TOK_W), jnp.float32),
            jax.ShapeDtypeStruct((_NW, NQ, TOK_W), jnp.float32),
        ],
    )(x, wt, b)
    probs_t, idx_t = _sc_topk(qmax, qsec)
    return (logits, probs_t.T, idx_t.T)
